# baseline (speedup 1.0000x reference)
import jax
import jax.numpy as jnp
from jax.experimental import pallas as pl
from jax.experimental.pallas import tpu as pltpu

_B = 32
_H = 64
_KD = 64
_NG = 5
_BLK = 8000


def _dot_t(a, b):
    return jax.lax.dot_general(a, b, (((1,), (1,)), ((), ())),
                               preferred_element_type=jnp.float32)


def _body(obs_ref, pin_ref, h0_ref, c0_ref, w1_ref, b1_ref, w2_ref, b2_ref,
          keys_ref, vals_ref, wih_ref, whh_ref, bih_ref, bhh_ref,
          hw_ref, hb_ref,
          head_out, h_out, c_out, feats_out,
          feats_s, m_s, l_s, acc_s):
    i = pl.program_id(0)
    nblk = pl.num_programs(0)

    @pl.when(i == 0)
    def _init():
        h1 = jnp.maximum(_dot_t(obs_ref[...], w1_ref[...]) + b1_ref[...], 0.0)
        f = jnp.maximum(_dot_t(h1, w2_ref[...]) + b2_ref[...], 0.0)
        feats_s[...] = f
        feats_out[...] = f
        m_s[...] = jnp.full((_B, 1), -jnp.inf, jnp.float32)
        l_s[...] = jnp.zeros((_B, 1), jnp.float32)
        acc_s[...] = jnp.zeros((_B, _H), jnp.float32)

    feats = feats_s[...]
    keys = keys_ref[...]
    ones_row = jnp.ones((1, _KD), jnp.float32)
    k2 = _dot_t(ones_row, keys * keys)
    s = 2.0 * _dot_t(feats, keys) - k2
    m_prev = m_s[...]
    m_cur = jnp.maximum(m_prev, jnp.max(s, axis=1, keepdims=True))
    alpha = jnp.exp(m_prev - m_cur)
    p = jnp.exp(s - m_cur)
    l_s[...] = alpha * l_s[...] + jnp.sum(p, axis=1, keepdims=True)
    acc_s[...] = alpha * acc_s[...] + jax.lax.dot_general(
        p, vals_ref[...], (((1,), (0,)), ((), ())),
        preferred_element_type=jnp.float32)
    m_s[...] = m_cur

    @pl.when(i == nblk - 1)
    def _final():
        m_t = acc_s[...] / l_s[...]
        x_t = jnp.concatenate([feats_s[...], pin_ref[...]], axis=1)
        gx = _dot_t(x_t, wih_ref[...]) + bih_ref[...] + bhh_ref[...]
        h = h0_ref[...]
        c = c0_ref[...]
        for t in range(_B):
            g = gx[t:t + 1, :] + _dot_t(h, whh_ref[...])
            gi = g[:, 0 * _H:1 * _H]
            gf = g[:, 1 * _H:2 * _H]
            gg = g[:, 2 * _H:3 * _H]
            go = g[:, 3 * _H:4 * _H]
            gr = g[:, 4 * _H:5 * _H]
            c = (jax.nn.sigmoid(gf) * c + jax.nn.sigmoid(gi) * jnp.tanh(gg)
                 + jax.nn.sigmoid(gr) * m_t[t:t + 1, :])
            h = jax.nn.sigmoid(go) * jnp.tanh(c)
        head_out[...] = _dot_t(h, hw_ref[...]) + hb_ref[...]
        h_out[...] = h
        c_out[...] = c


def kernel(obs, p_input, h0, c0, enc_W1, enc_b1, enc_W2, enc_b2,
           dnd_keys, dnd_vals, W_ih, W_hh, b_ih, b_hh,
           actor_W, actor_b, critic_W, critic_b):
    dl, kd = dnd_keys.shape
    assert dl % _BLK == 0
    nblk = dl // _BLK
    na = actor_W.shape[0]
    b = obs.shape[0]

    fixed = pl.BlockSpec(index_map=lambda i: (0, 0))
    grid = (nblk,)
    in_specs = [
            fixed,
            fixed,
            fixed, fixed,
            fixed, fixed, fixed, fixed,
            pl.BlockSpec((_BLK, kd), lambda i: (i, 0)),
            pl.BlockSpec((_BLK, _H), lambda i: (i, 0)),
            fixed, fixed, fixed, fixed,
            fixed, fixed,
    ]
    out_specs = [fixed, fixed, fixed, fixed]
    out_type = [
        jax.ShapeDtypeStruct((1, na + 1), jnp.float32),
        jax.ShapeDtypeStruct((1, _H), jnp.float32),
        jax.ShapeDtypeStruct((1, _H), jnp.float32),
        jax.ShapeDtypeStruct((b, _H), jnp.float32),
    ]
    scratch = [
        pltpu.VMEM((b, _H), jnp.float32),
        pltpu.VMEM((b, 1), jnp.float32),
        pltpu.VMEM((b, 1), jnp.float32),
        pltpu.VMEM((b, _H), jnp.float32),
    ]
    head_W = jnp.concatenate([actor_W, critic_W], axis=0)
    head_b = jnp.concatenate([actor_b, critic_b])[None, :]
    head, h_t, c_t, feats = pl.pallas_call(
        _body,
        grid=grid,
        in_specs=in_specs,
        out_specs=out_specs,
        out_shape=out_type,
        scratch_shapes=scratch,
        compiler_params=pltpu.CompilerParams(
            dimension_semantics=("arbitrary",),
        ),
    )(obs, p_input, h0.reshape(1, _H), c0.reshape(1, _H),
      enc_W1, enc_b1.reshape(1, -1), enc_W2, enc_b2.reshape(1, -1),
      dnd_keys, dnd_vals,
      W_ih, W_hh, b_ih.reshape(1, -1), b_hh.reshape(1, -1),
      head_W, head_b)
    return (head[:, :na].reshape(1, 1, na), head[:, na:].reshape(1, 1, 1),
            h_t.reshape(1, 1, _H), c_t.reshape(1, 1, _H), feats)
